# Initial kernel scaffold; baseline (speedup 1.0000x reference)
import jax
import jax.numpy as jnp
from jax.experimental import pallas as pl


def kernel(tokens, table, W_proj, b_proj, W_ih, W_hh, b_ih, b_hh):
    raise NotImplementedError("write your pallas kernel here")


# speedup vs baseline: 1.3056x; 1.3056x over previous
import functools

import jax
import jax.numpy as jnp
from jax import lax
from jax.experimental import pallas as pl
from jax.experimental.pallas import tpu as pltpu
from jax.experimental.pallas import tpu_sc as plsc

_D_EMB = 16
_D_MODEL = 128
_G = 4 * _D_MODEL
_PAD = 0
_L = 50
_LP = 56
_TS = 8
_NB = _LP // _TS
_NW = 32
_ROWS = 125000
_CHUNK = 448


def _gather_rows(table128, idx8):
    n = idx8.shape[0]
    b_per_w = n // _NW
    n_chunks = b_per_w // _CHUNK
    mesh = plsc.VectorSubcoreMesh(core_axis_name="c", subcore_axis_name="s")

    @functools.partial(
        pl.kernel,
        mesh=mesh,
        out_type=jax.ShapeDtypeStruct((n, 128), table128.dtype),
        scratch_types=[
            pltpu.VMEM((b_per_w,), jnp.int32),
            pltpu.VMEM((_CHUNK, 128), jnp.float32),
            pltpu.SemaphoreType.DMA,
        ],
    )
    def k(table_hbm, idx_hbm, out_hbm, idx_v, rows_v, sem):
        wid = lax.axis_index("s") * 2 + lax.axis_index("c")
        base = wid * b_per_w
        pltpu.sync_copy(idx_hbm.at[pl.ds(base, b_per_w)], idx_v)
        for ch in range(n_chunks):
            pltpu.async_copy(
                table_hbm.at[idx_v.at[pl.ds(ch * _CHUNK, _CHUNK)]],
                rows_v, sem).wait()
            pltpu.sync_copy(rows_v,
                            out_hbm.at[pl.ds(base + ch * _CHUNK, _CHUNK)])

    return k(table128, idx8)


def _lstm_body(emb_ref, sel_ref, wp_ref, wih_ref, whh_ref, bp_ref, bsum_ref,
               out_ref, h_ref, c_ref):
    tb = pl.program_id(0)

    @pl.when(tb == 0)
    def _():
        h_ref[...] = jnp.zeros_like(h_ref)
        c_ref[...] = jnp.zeros_like(c_ref)

    wih = wih_ref[...]
    wc = jnp.dot(wp_ref[...], wih,
                 preferred_element_type=jnp.float32)
    ws = jnp.concatenate([wc] * 8, axis=0)
    bc = jnp.dot(bp_ref[...], wih,
                 preferred_element_type=jnp.float32) + bsum_ref[...]
    whh = whh_ref[...]
    h = h_ref[...]
    c = c_ref[...]
    batch = h.shape[0]
    lane_grp = jax.lax.broadcasted_iota(jnp.int32, (batch, 128), 1) // 16
    for j in range(_TS):
        x = emb_ref[j]
        sel = sel_ref[j][:, None]
        xm = jnp.where(lane_grp == sel, x, 0.0)
        gates = (jnp.dot(xm, ws, preferred_element_type=jnp.float32)
                 + jnp.dot(h, whh, preferred_element_type=jnp.float32)
                 + bc)
        i = jax.nn.sigmoid(gates[:, 0:_D_MODEL])
        f = jax.nn.sigmoid(gates[:, _D_MODEL:2 * _D_MODEL])
        g = jnp.tanh(gates[:, 2 * _D_MODEL:3 * _D_MODEL])
        o = jax.nn.sigmoid(gates[:, 3 * _D_MODEL:4 * _D_MODEL])
        c = f * c + i * g
        h = o * jnp.tanh(c)
        out_ref[j] = h
    h_ref[...] = h
    c_ref[...] = c


def _lstm(emb, sel, W_proj, wih_t, whh_t, bp, bsum, batch):
    return pl.pallas_call(
        _lstm_body,
        grid=(_NB,),
        in_specs=[
            pl.BlockSpec((_TS, batch, 128), lambda tb: (tb, 0, 0)),
            pl.BlockSpec((_TS, batch), lambda tb: (tb, 0)),
            pl.BlockSpec((_D_EMB, _D_MODEL), lambda tb: (0, 0)),
            pl.BlockSpec((_D_MODEL, _G), lambda tb: (0, 0)),
            pl.BlockSpec((_D_MODEL, _G), lambda tb: (0, 0)),
            pl.BlockSpec((1, _D_MODEL), lambda tb: (0, 0)),
            pl.BlockSpec((1, _G), lambda tb: (0, 0)),
        ],
        out_specs=pl.BlockSpec((_TS, batch, _D_MODEL), lambda tb: (tb, 0, 0)),
        out_shape=jax.ShapeDtypeStruct((_LP, batch, _D_MODEL), jnp.float32),
        scratch_shapes=[
            pltpu.VMEM((batch, _D_MODEL), jnp.float32),
            pltpu.VMEM((batch, _D_MODEL), jnp.float32),
        ],
        compiler_params=pltpu.CompilerParams(
            dimension_semantics=("arbitrary",)),
    )(emb, sel, W_proj, wih_t, whh_t, bp, bsum)


def kernel(tokens, table, W_proj, b_proj, W_ih, W_hh, b_ih, b_hh):
    batch, seq_len = tokens.shape
    padding_masks = (tokens == _PAD)[:, None, None, :]
    sequential_masks = jnp.triu(jnp.ones((seq_len, seq_len), dtype=bool), k=1)

    tokens_pad = jnp.pad(tokens, ((0, 0), (0, _LP - _L))).T
    idx8 = (tokens_pad // 8).reshape(-1)
    sel = tokens_pad % 8
    table128 = table.reshape(_ROWS, 128)
    emb = _gather_rows(table128, idx8).reshape(_LP, batch, 128)

    out = _lstm(emb, sel, W_proj, W_ih.T, W_hh.T,
                b_proj.reshape(1, _D_MODEL),
                (b_ih + b_hh).reshape(1, _G), batch)
    features = jnp.swapaxes(out, 0, 1)[:, :_L, :]
    return (features, (padding_masks, sequential_masks))


# speedup vs baseline: 1.6418x; 1.2575x over previous
import dataclasses
import functools

import jax
import jax.numpy as jnp
from jax import lax
from jax.experimental import pallas as pl
from jax.experimental.pallas import tpu as pltpu
from jax.experimental.pallas import tpu_sc as plsc

_D_EMB = 16
_D_MODEL = 128
_G = 4 * _D_MODEL
_PAD = 0
_L = 50
_LP = 56
_TS = 8
_NB = _LP // _TS
_NW = 32
_GROUPS = 125000
_CHUNK = 448


def _gather_rows(table, toks):
    n = toks.shape[0]
    b_per_w = n // _NW
    mesh = plsc.VectorSubcoreMesh(core_axis_name="c", subcore_axis_name="s")

    @functools.partial(
        pl.kernel,
        mesh=mesh,
        out_type=jax.ShapeDtypeStruct((n, _D_EMB), jnp.float32),
        scratch_types=[
            pltpu.VMEM((b_per_w,), jnp.int32),
            pltpu.VMEM((_CHUNK, _D_EMB), jnp.float32),
            pltpu.SemaphoreType.DMA,
        ],
    )
    def k(tab_hbm, tok_hbm, out_hbm, tok_v, out_v, sem):
        wid = lax.axis_index("s") * 2 + lax.axis_index("c")
        base = wid * b_per_w
        pltpu.sync_copy(tok_hbm.at[pl.ds(base, b_per_w)], tok_v)

        for ch in range(b_per_w // _CHUNK):
            @pl.loop(0, _CHUNK // 16)
            def _(iv):
                tvec = tok_v[pl.ds(ch * _CHUNK + iv * 16, 16)]
                for j in range(16):
                    t = tvec[j]
                    pltpu.async_copy(tab_hbm.at[pl.ds(t, 1)],
                                     out_v.at[pl.ds(iv * 16 + j, 1)], sem)

            pltpu.make_async_copy(tab_hbm.at[pl.ds(0, _CHUNK)],
                                  out_v, sem).wait()
            pltpu.sync_copy(out_v,
                            out_hbm.at[pl.ds(base + ch * _CHUNK, _CHUNK)])

    return k(table, toks)


def _lstm_body(emb_ref, wp_ref, wih_ref, whh_ref, bp_ref, bsum_ref,
               out_ref, h_ref, c_ref):
    tb = pl.program_id(0)

    @pl.when(tb == 0)
    def _():
        h_ref[...] = jnp.zeros_like(h_ref)
        c_ref[...] = jnp.zeros_like(c_ref)

    wih = wih_ref[...]
    wc = jnp.dot(wp_ref[...], wih,
                 preferred_element_type=jnp.float32)
    bc = jnp.dot(bp_ref[...], wih,
                 preferred_element_type=jnp.float32) + bsum_ref[...]
    whh = whh_ref[...]
    h = h_ref[...]
    c = c_ref[...]
    for j in range(_TS):
        x = emb_ref[j]
        gates = (jnp.dot(x, wc, preferred_element_type=jnp.float32)
                 + jnp.dot(h, whh, preferred_element_type=jnp.float32)
                 + bc)
        i = jax.nn.sigmoid(gates[:, 0:_D_MODEL])
        f = jax.nn.sigmoid(gates[:, _D_MODEL:2 * _D_MODEL])
        g = jnp.tanh(gates[:, 2 * _D_MODEL:3 * _D_MODEL])
        o = jax.nn.sigmoid(gates[:, 3 * _D_MODEL:4 * _D_MODEL])
        c = f * c + i * g
        h = o * jnp.tanh(c)
        out_ref[:, j, :] = h
    h_ref[...] = h
    c_ref[...] = c


def _lstm(emb, W_proj, wih_t, whh_t, bp, bsum, batch):
    return pl.pallas_call(
        _lstm_body,
        grid=(_NB,),
        in_specs=[
            pl.BlockSpec((_TS, batch, _D_EMB), lambda tb: (tb, 0, 0)),
            pl.BlockSpec((_D_EMB, _D_MODEL), lambda tb: (0, 0)),
            pl.BlockSpec((_D_MODEL, _G), lambda tb: (0, 0)),
            pl.BlockSpec((_D_MODEL, _G), lambda tb: (0, 0)),
            pl.BlockSpec((1, _D_MODEL), lambda tb: (0, 0)),
            pl.BlockSpec((1, _G), lambda tb: (0, 0)),
        ],
        out_specs=pl.BlockSpec((batch, _TS, _D_MODEL), lambda tb: (0, tb, 0)),
        out_shape=jax.ShapeDtypeStruct((batch, _LP, _D_MODEL), jnp.float32),
        scratch_shapes=[
            pltpu.VMEM((batch, _D_MODEL), jnp.float32),
            pltpu.VMEM((batch, _D_MODEL), jnp.float32),
        ],
        compiler_params=pltpu.CompilerParams(
            dimension_semantics=("arbitrary",)),
    )(emb, W_proj, wih_t, whh_t, bp, bsum)


def kernel(tokens, table, W_proj, b_proj, W_ih, W_hh, b_ih, b_hh):
    batch, seq_len = tokens.shape
    padding_masks = (tokens == _PAD)[:, None, None, :]
    sequential_masks = jnp.triu(jnp.ones((seq_len, seq_len), dtype=bool), k=1)

    toks = jnp.pad(tokens, ((0, 0), (0, _LP - _L))).T.reshape(-1)
    emb = _gather_rows(table, toks).reshape(_LP, batch, _D_EMB)

    out = _lstm(emb, W_proj, W_ih.T, W_hh.T,
                b_proj.reshape(1, _D_MODEL),
                (b_ih + b_hh).reshape(1, _G), batch)
    features = out[:, :_L, :]
    return (features, (padding_masks, sequential_masks))


# speedup vs baseline: 2.7632x; 1.6830x over previous
import functools

import jax
import jax.numpy as jnp
from jax import lax
from jax.experimental import pallas as pl
from jax.experimental.pallas import tpu as pltpu
from jax.experimental.pallas import tpu_sc as plsc

_D_EMB = 16
_D_MODEL = 128
_G = 4 * _D_MODEL
_PAD = 0
_L = 50
_TS = 8
_NB = 7
_NW = 32
_CHUNK = 400


_RB = 4096
_NRB = 245


def _relayout_body(tc_ref, out_ref):
    out_ref[...] = tc_ref[...].T


def _relayout(table_cols):
    v = table_cols.shape[1]
    return pl.pallas_call(
        _relayout_body,
        grid=(_NRB,),
        in_specs=[pl.BlockSpec((_D_EMB, _RB), lambda i: (0, i))],
        out_specs=pl.BlockSpec((_RB, _D_EMB), lambda i: (i, 0)),
        out_shape=jax.ShapeDtypeStruct((v, _D_EMB), jnp.float32),
    )(table_cols)


def _gather_rows(table, toks):
    n = toks.shape[0]
    b_per_w = n // _NW
    mesh = plsc.VectorSubcoreMesh(core_axis_name="c", subcore_axis_name="s")

    @functools.partial(
        pl.kernel,
        mesh=mesh,
        out_type=jax.ShapeDtypeStruct((n, _D_EMB), jnp.float32),
        scratch_types=[
            pltpu.VMEM((b_per_w,), jnp.int32),
            pltpu.VMEM((_CHUNK, _D_EMB), jnp.float32),
            pltpu.SemaphoreType.DMA,
        ],
    )
    def k(tab_hbm, tok_hbm, out_hbm, tok_v, out_v, sem):
        wid = lax.axis_index("s") * 2 + lax.axis_index("c")
        base = wid * b_per_w
        pltpu.sync_copy(tok_hbm.at[pl.ds(base, b_per_w)], tok_v)

        for ch in range(b_per_w // _CHUNK):
            @pl.loop(0, _CHUNK // 16)
            def _(iv):
                tvec = tok_v[pl.ds(ch * _CHUNK + iv * 16, 16)]
                for j in range(16):
                    t = tvec[j]
                    pltpu.async_copy(tab_hbm.at[pl.ds(t, 1)],
                                     out_v.at[pl.ds(iv * 16 + j, 1)], sem)

            pltpu.make_async_copy(tab_hbm.at[pl.ds(0, _CHUNK)],
                                  out_v, sem).wait()
            pltpu.sync_copy(out_v,
                            out_hbm.at[pl.ds(base + ch * _CHUNK, _CHUNK)])

    return k(table, toks)


def _lstm_body(emb_ref, wp_ref, wih_ref, whh_ref, bp_ref, bsum_ref,
               out_ref, h_ref, c_ref):
    tb = pl.program_id(0)

    @pl.when(tb == 0)
    def _():
        h_ref[...] = jnp.zeros_like(h_ref)
        c_ref[...] = jnp.zeros_like(c_ref)

    wih = wih_ref[...]
    wc = jnp.dot(wp_ref[...], wih,
                 preferred_element_type=jnp.float32)
    bc = jnp.dot(bp_ref[...], wih,
                 preferred_element_type=jnp.float32) + bsum_ref[...]
    whh = whh_ref[...]
    h = h_ref[...]
    c = c_ref[...]
    for j in range(_TS):
        x = emb_ref[j]
        gates = (jnp.dot(x, wc, preferred_element_type=jnp.float32)
                 + jnp.dot(h, whh, preferred_element_type=jnp.float32)
                 + bc)
        i = jax.nn.sigmoid(gates[:, 0:_D_MODEL])
        f = jax.nn.sigmoid(gates[:, _D_MODEL:2 * _D_MODEL])
        g = jnp.tanh(gates[:, 2 * _D_MODEL:3 * _D_MODEL])
        o = jax.nn.sigmoid(gates[:, 3 * _D_MODEL:4 * _D_MODEL])
        c = f * c + i * g
        h = o * jnp.tanh(c)
        out_ref[j] = h
    h_ref[...] = h
    c_ref[...] = c


def _lstm(emb, W_proj, wih_t, whh_t, bp, bsum, batch):
    return pl.pallas_call(
        _lstm_body,
        grid=(_NB,),
        in_specs=[
            pl.BlockSpec((_TS, batch, _D_EMB), lambda tb: (tb, 0, 0)),
            pl.BlockSpec((_D_EMB, _D_MODEL), lambda tb: (0, 0)),
            pl.BlockSpec((_D_MODEL, _G), lambda tb: (0, 0)),
            pl.BlockSpec((_D_MODEL, _G), lambda tb: (0, 0)),
            pl.BlockSpec((1, _D_MODEL), lambda tb: (0, 0)),
            pl.BlockSpec((1, _G), lambda tb: (0, 0)),
        ],
        out_specs=pl.BlockSpec((_TS, batch, _D_MODEL), lambda tb: (tb, 0, 0)),
        out_shape=jax.ShapeDtypeStruct((_L, batch, _D_MODEL), jnp.float32),
        scratch_shapes=[
            pltpu.VMEM((batch, _D_MODEL), jnp.float32),
            pltpu.VMEM((batch, _D_MODEL), jnp.float32),
        ],
        compiler_params=pltpu.CompilerParams(
            dimension_semantics=("arbitrary",)),
    )(emb, W_proj, wih_t, whh_t, bp, bsum)


def kernel(tokens, table, W_proj, b_proj, W_ih, W_hh, b_ih, b_hh):
    batch, seq_len = tokens.shape
    padding_masks = (tokens == _PAD)[:, None, None, :]
    sequential_masks = jnp.triu(jnp.ones((seq_len, seq_len), dtype=bool), k=1)

    toks = tokens.T.reshape(-1)
    table_rm = _relayout(jnp.swapaxes(table, 0, 1))
    emb = _gather_rows(table_rm, toks).reshape(_L, batch, _D_EMB)

    out = _lstm(emb, W_proj, W_ih.T, W_hh.T,
                b_proj.reshape(1, _D_MODEL),
                (b_ih + b_hh).reshape(1, _G), batch)
    features = jnp.swapaxes(out, 0, 1)
    return (features, (padding_masks, sequential_masks))


# speedup vs baseline: 3.7916x; 1.3722x over previous
import functools

import jax
import jax.numpy as jnp
from jax import lax
from jax.experimental import pallas as pl
from jax.experimental.pallas import tpu as pltpu
from jax.experimental.pallas import tpu_sc as plsc

_D_EMB = 16
_D_MODEL = 128
_G = 4 * _D_MODEL
_PAD = 0
_L = 50
_TS = 8
_NB = 7
_NW = 32
_CHUNK = 400


_RB = 16384
_NRB = 62


def _relayout_body(tc_ref, out_ref):
    out_ref[...] = tc_ref[...].T


def _relayout(table_cols):
    v = table_cols.shape[1]
    return pl.pallas_call(
        _relayout_body,
        grid=(_NRB,),
        in_specs=[pl.BlockSpec((_D_EMB, _RB), lambda i: (0, i))],
        out_specs=pl.BlockSpec((_RB, _D_EMB), lambda i: (i, 0)),
        out_shape=jax.ShapeDtypeStruct((v, _D_EMB), jnp.float32),
    )(table_cols)


def _gather_rows(table, toks):
    n = toks.shape[0]
    b_per_w = n // _NW
    mesh = plsc.VectorSubcoreMesh(core_axis_name="c", subcore_axis_name="s")

    @functools.partial(
        pl.kernel,
        mesh=mesh,
        out_type=jax.ShapeDtypeStruct((n, _D_EMB), jnp.float32),
        scratch_types=[
            pltpu.VMEM((b_per_w,), jnp.int32),
            pltpu.VMEM((_CHUNK, _D_EMB), jnp.float32),
            pltpu.SemaphoreType.DMA,
        ],
    )
    def k(tab_hbm, tok_hbm, out_hbm, tok_v, out_v, sem):
        wid = lax.axis_index("s") * 2 + lax.axis_index("c")
        base = wid * b_per_w
        pltpu.sync_copy(tok_hbm.at[pl.ds(base, b_per_w)], tok_v)

        for ch in range(b_per_w // _CHUNK):
            @pl.loop(0, _CHUNK // 16)
            def _(iv):
                tvec = tok_v[pl.ds(ch * _CHUNK + iv * 16, 16)]
                for j in range(16):
                    t = tvec[j]
                    pltpu.async_copy(tab_hbm.at[pl.ds(t, 1)],
                                     out_v.at[pl.ds(iv * 16 + j, 1)], sem)

            pltpu.make_async_copy(tab_hbm.at[pl.ds(0, _CHUNK)],
                                  out_v, sem).wait()
            pltpu.sync_copy(out_v,
                            out_hbm.at[pl.ds(base + ch * _CHUNK, _CHUNK)])

    return k(table, toks)


def _lstm_body(emb_ref, wp_ref, wih_ref, whh_ref, bp_ref, bsum_ref,
               out_ref, h_ref, c_ref):
    tb = pl.program_id(0)

    @pl.when(tb == 0)
    def _():
        h_ref[...] = jnp.zeros_like(h_ref)
        c_ref[...] = jnp.zeros_like(c_ref)

    wih = wih_ref[...]
    wc = jnp.dot(wp_ref[...], wih,
                 preferred_element_type=jnp.float32)
    bc = jnp.dot(bp_ref[...], wih,
                 preferred_element_type=jnp.float32) + bsum_ref[...]
    whh = whh_ref[...]
    h = h_ref[...]
    c = c_ref[...]
    for j in range(_TS):
        x = emb_ref[j]
        gates = (jnp.dot(x, wc, preferred_element_type=jnp.float32)
                 + jnp.dot(h, whh, preferred_element_type=jnp.float32)
                 + bc)
        i = jax.nn.sigmoid(gates[:, 0:_D_MODEL])
        f = jax.nn.sigmoid(gates[:, _D_MODEL:2 * _D_MODEL])
        g = jnp.tanh(gates[:, 2 * _D_MODEL:3 * _D_MODEL])
        o = jax.nn.sigmoid(gates[:, 3 * _D_MODEL:4 * _D_MODEL])
        c = f * c + i * g
        h = o * jnp.tanh(c)
        out_ref[j] = h
    h_ref[...] = h
    c_ref[...] = c


def _lstm(emb, W_proj, wih_t, whh_t, bp, bsum, batch):
    return pl.pallas_call(
        _lstm_body,
        grid=(_NB,),
        in_specs=[
            pl.BlockSpec((_TS, batch, _D_EMB), lambda tb: (tb, 0, 0)),
            pl.BlockSpec((_D_EMB, _D_MODEL), lambda tb: (0, 0)),
            pl.BlockSpec((_D_MODEL, _G), lambda tb: (0, 0)),
            pl.BlockSpec((_D_MODEL, _G), lambda tb: (0, 0)),
            pl.BlockSpec((1, _D_MODEL), lambda tb: (0, 0)),
            pl.BlockSpec((1, _G), lambda tb: (0, 0)),
        ],
        out_specs=pl.BlockSpec((_TS, batch, _D_MODEL), lambda tb: (tb, 0, 0)),
        out_shape=jax.ShapeDtypeStruct((_L, batch, _D_MODEL), jnp.float32),
        scratch_shapes=[
            pltpu.VMEM((batch, _D_MODEL), jnp.float32),
            pltpu.VMEM((batch, _D_MODEL), jnp.float32),
        ],
        compiler_params=pltpu.CompilerParams(
            dimension_semantics=("arbitrary",)),
    )(emb, W_proj, wih_t, whh_t, bp, bsum)


def kernel(tokens, table, W_proj, b_proj, W_ih, W_hh, b_ih, b_hh):
    batch, seq_len = tokens.shape
    padding_masks = (tokens == _PAD)[:, None, None, :]
    sequential_masks = jnp.triu(jnp.ones((seq_len, seq_len), dtype=bool), k=1)

    toks = tokens.T.reshape(-1)
    table_rm = _relayout(jnp.swapaxes(table, 0, 1))
    emb = _gather_rows(table_rm, toks).reshape(_L, batch, _D_EMB)

    out = _lstm(emb, W_proj, W_ih.T, W_hh.T,
                b_proj.reshape(1, _D_MODEL),
                (b_ih + b_hh).reshape(1, _G), batch)
    features = jnp.swapaxes(out, 0, 1)
    return (features, (padding_masks, sequential_masks))


# speedup vs baseline: 3.8327x; 1.0108x over previous
import functools

import jax
import jax.numpy as jnp
from jax import lax
from jax.experimental import pallas as pl
from jax.experimental.pallas import tpu as pltpu
from jax.experimental.pallas import tpu_sc as plsc

_D_EMB = 16
_D_MODEL = 128
_G = 4 * _D_MODEL
_PAD = 0
_L = 50
_TS = 8
_NB = 7
_NW = 32
_CHUNK = 400


_RB = 32768
_NRB = 31


def _relayout_body(tc_ref, out_ref):
    out_ref[...] = tc_ref[...].T


def _relayout(table_cols):
    v = table_cols.shape[1]
    return pl.pallas_call(
        _relayout_body,
        grid=(_NRB,),
        in_specs=[pl.BlockSpec((_D_EMB, _RB), lambda i: (0, i))],
        out_specs=pl.BlockSpec((_RB, _D_EMB), lambda i: (i, 0)),
        out_shape=jax.ShapeDtypeStruct((v, _D_EMB), jnp.float32),
    )(table_cols)


def _gather_rows(table, toks):
    n = toks.shape[0]
    b_per_w = n // _NW
    mesh = plsc.VectorSubcoreMesh(core_axis_name="c", subcore_axis_name="s")

    @functools.partial(
        pl.kernel,
        mesh=mesh,
        out_type=jax.ShapeDtypeStruct((n, _D_EMB), jnp.float32),
        scratch_types=[
            pltpu.VMEM((b_per_w,), jnp.int32),
            pltpu.VMEM((_CHUNK, _D_EMB), jnp.float32),
            pltpu.SemaphoreType.DMA,
        ],
    )
    def k(tab_hbm, tok_hbm, out_hbm, tok_v, out_v, sem):
        wid = lax.axis_index("s") * 2 + lax.axis_index("c")
        base = wid * b_per_w
        pltpu.sync_copy(tok_hbm.at[pl.ds(base, b_per_w)], tok_v)

        for ch in range(b_per_w // _CHUNK):
            @pl.loop(0, _CHUNK // 16)
            def _(iv):
                tvec = tok_v[pl.ds(ch * _CHUNK + iv * 16, 16)]
                for j in range(16):
                    t = tvec[j]
                    pltpu.async_copy(tab_hbm.at[pl.ds(t, 1)],
                                     out_v.at[pl.ds(iv * 16 + j, 1)], sem)

            pltpu.make_async_copy(tab_hbm.at[pl.ds(0, _CHUNK)],
                                  out_v, sem).wait()
            pltpu.sync_copy(out_v,
                            out_hbm.at[pl.ds(base + ch * _CHUNK, _CHUNK)])

    return k(table, toks)


def _lstm_body(emb_ref, wp_ref, wih_ref, whh_ref, bp_ref, bsum_ref,
               out_ref, h_ref, c_ref):
    tb = pl.program_id(0)

    @pl.when(tb == 0)
    def _():
        h_ref[...] = jnp.zeros_like(h_ref)
        c_ref[...] = jnp.zeros_like(c_ref)

    wih = wih_ref[...]
    wc = jnp.dot(wp_ref[...], wih,
                 preferred_element_type=jnp.float32)
    bc = jnp.dot(bp_ref[...], wih,
                 preferred_element_type=jnp.float32) + bsum_ref[...]
    whh = whh_ref[...]
    h = h_ref[...]
    c = c_ref[...]
    for j in range(_TS):
        x = emb_ref[j]
        gates = (jnp.dot(x, wc, preferred_element_type=jnp.float32)
                 + jnp.dot(h, whh, preferred_element_type=jnp.float32)
                 + bc)
        i = jax.nn.sigmoid(gates[:, 0:_D_MODEL])
        f = jax.nn.sigmoid(gates[:, _D_MODEL:2 * _D_MODEL])
        g = jnp.tanh(gates[:, 2 * _D_MODEL:3 * _D_MODEL])
        o = jax.nn.sigmoid(gates[:, 3 * _D_MODEL:4 * _D_MODEL])
        c = f * c + i * g
        h = o * jnp.tanh(c)
        out_ref[j] = h
    h_ref[...] = h
    c_ref[...] = c


def _lstm(emb, W_proj, wih_t, whh_t, bp, bsum, batch):
    return pl.pallas_call(
        _lstm_body,
        grid=(_NB,),
        in_specs=[
            pl.BlockSpec((_TS, batch, _D_EMB), lambda tb: (tb, 0, 0)),
            pl.BlockSpec((_D_EMB, _D_MODEL), lambda tb: (0, 0)),
            pl.BlockSpec((_D_MODEL, _G), lambda tb: (0, 0)),
            pl.BlockSpec((_D_MODEL, _G), lambda tb: (0, 0)),
            pl.BlockSpec((1, _D_MODEL), lambda tb: (0, 0)),
            pl.BlockSpec((1, _G), lambda tb: (0, 0)),
        ],
        out_specs=pl.BlockSpec((_TS, batch, _D_MODEL), lambda tb: (tb, 0, 0)),
        out_shape=jax.ShapeDtypeStruct((_L, batch, _D_MODEL), jnp.float32),
        scratch_shapes=[
            pltpu.VMEM((batch, _D_MODEL), jnp.float32),
            pltpu.VMEM((batch, _D_MODEL), jnp.float32),
        ],
        compiler_params=pltpu.CompilerParams(
            dimension_semantics=("arbitrary",)),
    )(emb, W_proj, wih_t, whh_t, bp, bsum)


def kernel(tokens, table, W_proj, b_proj, W_ih, W_hh, b_ih, b_hh):
    batch, seq_len = tokens.shape
    padding_masks = (tokens == _PAD)[:, None, None, :]
    sequential_masks = jnp.triu(jnp.ones((seq_len, seq_len), dtype=bool), k=1)

    toks = tokens.T.reshape(-1)
    table_rm = _relayout(jnp.swapaxes(table, 0, 1))
    emb = _gather_rows(table_rm, toks).reshape(_L, batch, _D_EMB)

    out = _lstm(emb, W_proj, W_ih.T, W_hh.T,
                b_proj.reshape(1, _D_MODEL),
                (b_ih + b_hh).reshape(1, _G), batch)
    features = jnp.swapaxes(out, 0, 1)
    return (features, (padding_masks, sequential_masks))


# speedup vs baseline: 3.8861x; 1.0139x over previous
import functools

import jax
import jax.numpy as jnp
from jax import lax
from jax.experimental import pallas as pl
from jax.experimental.pallas import tpu as pltpu
from jax.experimental.pallas import tpu_sc as plsc

_D_EMB = 16
_D_MODEL = 128
_G = 4 * _D_MODEL
_PAD = 0
_L = 50
_TS = 10
_NB = 5
_NW = 32
_CHUNK = 400


_RB = 32768
_NRB = 31


def _relayout_body(tc_ref, out_ref):
    out_ref[...] = tc_ref[...].T


def _relayout(table_cols):
    v = table_cols.shape[1]
    return pl.pallas_call(
        _relayout_body,
        grid=(_NRB,),
        in_specs=[pl.BlockSpec((_D_EMB, _RB), lambda i: (0, i))],
        out_specs=pl.BlockSpec((_RB, _D_EMB), lambda i: (i, 0)),
        out_shape=jax.ShapeDtypeStruct((v, _D_EMB), jnp.float32),
    )(table_cols)


def _gather_rows(table, toks):
    n = toks.shape[0]
    b_per_w = n // _NW
    mesh = plsc.VectorSubcoreMesh(core_axis_name="c", subcore_axis_name="s")

    @functools.partial(
        pl.kernel,
        mesh=mesh,
        out_type=jax.ShapeDtypeStruct((n, _D_EMB), jnp.float32),
        scratch_types=[
            pltpu.VMEM((b_per_w,), jnp.int32),
            pltpu.VMEM((_CHUNK, _D_EMB), jnp.float32),
            pltpu.SemaphoreType.DMA,
        ],
    )
    def k(tab_hbm, tok_hbm, out_hbm, tok_v, out_v, sem):
        wid = lax.axis_index("s") * 2 + lax.axis_index("c")
        base = wid * b_per_w
        pltpu.sync_copy(tok_hbm.at[pl.ds(base, b_per_w)], tok_v)

        for ch in range(b_per_w // _CHUNK):
            @pl.loop(0, _CHUNK // 16)
            def _(iv):
                tvec = tok_v[pl.ds(ch * _CHUNK + iv * 16, 16)]
                for j in range(16):
                    t = tvec[j]
                    pltpu.async_copy(tab_hbm.at[pl.ds(t, 1)],
                                     out_v.at[pl.ds(iv * 16 + j, 1)], sem)

            pltpu.make_async_copy(tab_hbm.at[pl.ds(0, _CHUNK)],
                                  out_v, sem).wait()
            pltpu.sync_copy(out_v,
                            out_hbm.at[pl.ds(base + ch * _CHUNK, _CHUNK)])

    return k(table, toks)


def _lstm_body(emb_ref, wp_ref, wih_ref, whh_ref, bp_ref, bsum_ref,
               out_ref, h_ref, c_ref):
    tb = pl.program_id(0)

    @pl.when(tb == 0)
    def _():
        h_ref[...] = jnp.zeros_like(h_ref)
        c_ref[...] = jnp.zeros_like(c_ref)

    wih = wih_ref[...]
    wc = jnp.dot(wp_ref[...], wih,
                 preferred_element_type=jnp.float32)
    bc = jnp.dot(bp_ref[...], wih,
                 preferred_element_type=jnp.float32) + bsum_ref[...]
    whh = whh_ref[...]
    h = h_ref[...]
    c = c_ref[...]
    for j in range(_TS):
        x = emb_ref[j]
        gates = (jnp.dot(x, wc, preferred_element_type=jnp.float32)
                 + jnp.dot(h, whh, preferred_element_type=jnp.float32)
                 + bc)
        i = jax.nn.sigmoid(gates[:, 0:_D_MODEL])
        f = jax.nn.sigmoid(gates[:, _D_MODEL:2 * _D_MODEL])
        g = jnp.tanh(gates[:, 2 * _D_MODEL:3 * _D_MODEL])
        o = jax.nn.sigmoid(gates[:, 3 * _D_MODEL:4 * _D_MODEL])
        c = f * c + i * g
        h = o * jnp.tanh(c)
        out_ref[j] = h
    h_ref[...] = h
    c_ref[...] = c


def _lstm(emb, W_proj, wih_t, whh_t, bp, bsum, batch):
    return pl.pallas_call(
        _lstm_body,
        grid=(_NB,),
        in_specs=[
            pl.BlockSpec((_TS, batch, _D_EMB), lambda tb: (tb, 0, 0)),
            pl.BlockSpec((_D_EMB, _D_MODEL), lambda tb: (0, 0)),
            pl.BlockSpec((_D_MODEL, _G), lambda tb: (0, 0)),
            pl.BlockSpec((_D_MODEL, _G), lambda tb: (0, 0)),
            pl.BlockSpec((1, _D_MODEL), lambda tb: (0, 0)),
            pl.BlockSpec((1, _G), lambda tb: (0, 0)),
        ],
        out_specs=pl.BlockSpec((_TS, batch, _D_MODEL), lambda tb: (tb, 0, 0)),
        out_shape=jax.ShapeDtypeStruct((_L, batch, _D_MODEL), jnp.float32),
        scratch_shapes=[
            pltpu.VMEM((batch, _D_MODEL), jnp.float32),
            pltpu.VMEM((batch, _D_MODEL), jnp.float32),
        ],
        compiler_params=pltpu.CompilerParams(
            dimension_semantics=("arbitrary",)),
    )(emb, W_proj, wih_t, whh_t, bp, bsum)


def kernel(tokens, table, W_proj, b_proj, W_ih, W_hh, b_ih, b_hh):
    batch, seq_len = tokens.shape
    padding_masks = (tokens == _PAD)[:, None, None, :]
    sequential_masks = jnp.triu(jnp.ones((seq_len, seq_len), dtype=bool), k=1)

    toks = tokens.T.reshape(-1)
    table_rm = _relayout(jnp.swapaxes(table, 0, 1))
    emb = _gather_rows(table_rm, toks).reshape(_L, batch, _D_EMB)

    out = _lstm(emb, W_proj, W_ih.T, W_hh.T,
                b_proj.reshape(1, _D_MODEL),
                (b_ih + b_hh).reshape(1, _G), batch)
    features = jnp.swapaxes(out, 0, 1)
    return (features, (padding_masks, sequential_masks))
